# Initial kernel scaffold; baseline (speedup 1.0000x reference)
import jax
import jax.numpy as jnp
from jax.experimental import pallas as pl


def kernel(x_author, x_paper, x_term, x_conf, ei_ap, ei_pa, ei_pt, ei_tp, ei_pc, ei_cp, Win, bin_, Wk, bk, Wq, bq, Wv, bv, Wa, ba, skip, a_rel, m_rel, p_rel, Wout, bout):
    raise NotImplementedError("write your pallas kernel here")


# speedup vs baseline: 1.8817x; 1.8817x over previous
import functools

import jax
import jax.numpy as jnp
import numpy as np
from jax.experimental import pallas as pl
from jax.experimental.pallas import tpu as pltpu

_NODE_TYPES = ['author', 'paper', 'term', 'conf']
_NNODES = {'author': 10000, 'paper': 20000, 'term': 4000, 'conf': 20}
_EDGE_TYPES = [('author', 'paper'), ('paper', 'author'), ('paper', 'term'),
               ('term', 'paper'), ('paper', 'conf'), ('conf', 'paper')]
_DIN = 256
_HID = 256
_HEADS = 8
_DH = _HID // _HEADS
_NLAYERS = 2


def _erf(x):
    s = jnp.sign(x)
    ax = jnp.abs(x)
    t = 1.0 / (1.0 + 0.3275911 * ax)
    poly = t * (0.254829592 + t * (-0.284496736 + t * (1.421413741
               + t * (-1.453152027 + t * 1.061405429))))
    return s * (1.0 - poly * jnp.exp(-ax * ax))


def _gelu(x):
    return 0.5 * x * (1.0 + _erf(x * np.float32(1.0 / np.sqrt(2.0))))


def _mm_body(x_ref, w_ref, b_ref, o_ref, *, act):
    y = jnp.dot(x_ref[...], w_ref[...], preferred_element_type=jnp.float32)
    y = y + b_ref[...]
    if act == 'relu':
        y = jnp.maximum(y, 0.0)
    elif act == 'gelu':
        y = _gelu(y)
    o_ref[...] = y


def _mm(x, w, b, act=None, bn=512):
    n, k = x.shape
    m = w.shape[1]
    npad = -(-n // bn) * bn
    if npad != n:
        x = jnp.pad(x, ((0, npad - n), (0, 0)))
    out = pl.pallas_call(
        functools.partial(_mm_body, act=act),
        grid=(npad // bn,),
        in_specs=[
            pl.BlockSpec((bn, k), lambda i: (i, 0)),
            pl.BlockSpec((k, m), lambda i: (0, 0)),
            pl.BlockSpec((1, m), lambda i: (0, 0)),
        ],
        out_specs=pl.BlockSpec((bn, m), lambda i: (i, 0)),
        out_shape=jax.ShapeDtypeStruct((npad, m), jnp.float32),
    )(x, w, b.reshape(1, m))
    return out[:n] if npad != n else out


def _blockdiag(a):
    eye = jnp.eye(_HEADS, dtype=a.dtype)
    return (eye[:, None, :, None] * a[:, :, None, :]).reshape(_HID, _HID)


def _edge_pass(kt, vt, q_dst, src, dst, p, n_dst):
    k_j = kt[src].reshape(-1, _HEADS, _DH)
    v_j = vt[src].reshape(-1, _HEADS, _DH)
    q_i = q_dst[dst].reshape(-1, _HEADS, _DH)
    alpha = (q_i * k_j).sum(-1) * p / np.sqrt(_DH)
    w = jnp.exp(alpha)
    num = jax.ops.segment_sum((v_j * w[:, :, None]).reshape(-1, _HID), dst, n_dst)
    den = jax.ops.segment_sum(w, dst, n_dst)
    return num, den


def _upd_body(a_ref, nums_ref, dens_ref, x_ref, wa_ref, ba_ref, o_ref, *, nsrc):
    row = jax.lax.broadcasted_iota(jnp.int32, (128, _HID), 0)
    col = jax.lax.broadcasted_iota(jnp.int32, (128, _HID), 1)
    expand = (col // _DH == row).astype(jnp.float32)
    agg = jnp.zeros((nums_ref.shape[0], _HID), jnp.float32)
    for s in range(nsrc):
        num = nums_ref[:, s * _HID:(s + 1) * _HID]
        den = dens_ref[:, s * 128:(s + 1) * 128]
        denb = jnp.dot(den, expand, preferred_element_type=jnp.float32)
        recip = jnp.where(denb > 0, 1.0 / jnp.maximum(denb, 1e-30), 0.0)
        agg = agg + num * recip
    o = jnp.dot(_gelu(agg), wa_ref[...],
                preferred_element_type=jnp.float32) + ba_ref[...]
    a = a_ref[0]
    o_ref[...] = a * o + (1.0 - a) * x_ref[...]


def _update(numdens, x, wa, ba, a_gate, bn=512):
    n = x.shape[0]
    nsrc = len(numdens)
    nums = jnp.concatenate([nu for (nu, _) in numdens], axis=1)
    dens = jnp.concatenate(
        [jnp.pad(de, ((0, 0), (0, 128 - _HEADS))) for (_, de) in numdens], axis=1)
    npad = -(-n // bn) * bn
    if npad != n:
        nums = jnp.pad(nums, ((0, npad - n), (0, 0)))
        dens = jnp.pad(dens, ((0, npad - n), (0, 0)))
        x = jnp.pad(x, ((0, npad - n), (0, 0)))
    out = pl.pallas_call(
        functools.partial(_upd_body, nsrc=nsrc),
        grid=(npad // bn,),
        in_specs=[
            pl.BlockSpec(memory_space=pltpu.SMEM),
            pl.BlockSpec((bn, nsrc * _HID), lambda i: (i, 0)),
            pl.BlockSpec((bn, nsrc * 128), lambda i: (i, 0)),
            pl.BlockSpec((bn, _HID), lambda i: (i, 0)),
            pl.BlockSpec((_HID, _HID), lambda i: (0, 0)),
            pl.BlockSpec((1, _HID), lambda i: (0, 0)),
        ],
        out_specs=pl.BlockSpec((bn, _HID), lambda i: (i, 0)),
        out_shape=jax.ShapeDtypeStruct((npad, _HID), jnp.float32),
    )(a_gate.reshape(1), nums, dens, x, wa, ba.reshape(1, _HID))
    return out[:n] if npad != n else out


def kernel(x_author, x_paper, x_term, x_conf, ei_ap, ei_pa, ei_pt, ei_tp,
           ei_pc, ei_cp, Win, bin_, Wk, bk, Wq, bq, Wv, bv, Wa, ba, skip,
           a_rel, m_rel, p_rel, Wout, bout):
    xs = {'author': x_author, 'paper': x_paper, 'term': x_term, 'conf': x_conf}
    eis = [ei_ap, ei_pa, ei_pt, ei_tp, ei_pc, ei_cp]

    x = {}
    for i, t in enumerate(_NODE_TYPES):
        x[t] = _mm(xs[t], Win[i], bin_[i], act='relu')

    for l in range(_NLAYERS):
        k = {}
        q = {}
        v = {}
        for i, t in enumerate(_NODE_TYPES):
            wkqv = jnp.concatenate([Wk[l, i], Wq[l, i], Wv[l, i]], axis=1)
            bkqv = jnp.concatenate([bk[l, i], bq[l, i], bv[l, i]], axis=0)
            kqv = _mm(x[t], wkqv, bkqv)
            k[t] = kqv[:, :_HID]
            q[t] = kqv[:, _HID:2 * _HID]
            v[t] = kqv[:, 2 * _HID:]

        numden = {t: [] for t in _NODE_TYPES}
        for r, (st, dt) in enumerate(_EDGE_TYPES):
            bda = _blockdiag(a_rel[l, r])
            bdm = _blockdiag(m_rel[l, r])
            ktvt = _mm(jnp.concatenate([k[st], v[st]], axis=0),
                       jnp.concatenate([bda, bdm], axis=1),
                       jnp.zeros((2 * _HID,), jnp.float32))
            ns = k[st].shape[0]
            kt = ktvt[:ns, :_HID]
            vt = ktvt[ns:, _HID:]
            num, den = _edge_pass(kt, vt, q[dt], eis[r][0], eis[r][1],
                                  p_rel[l, r], _NNODES[dt])
            numden[dt].append((num, den))

        newx = {}
        for i, t in enumerate(_NODE_TYPES):
            a_gate = jax.nn.sigmoid(skip[l, i])
            newx[t] = _update(numden[t], x[t], Wa[l, i], ba[l, i], a_gate)
        x = newx

    return _mm(x['author'], Wout, bout)
